# baseline (speedup 1.0000x reference)
import functools

import jax
import jax.numpy as jnp
from jax import lax
from jax.experimental import pallas as pl
from jax.experimental.pallas import tpu as pltpu
from jax.experimental.pallas import tpu_sc as plsc

N = 10000
E = 160000
D = 256
T = 6
NE_T = 5
B = 10
G = 1000
HALF = 5000
HALFP = 5120
CK = 800
ECK = 80
NCH = E // CK
NPAD = 10240

_mesh = lambda: plsc.VectorSubcoreMesh(core_axis_name="c", subcore_axis_name="s")


@functools.partial(
    pl.kernel,
    mesh=_mesh(),
    out_type=jax.ShapeDtypeStruct((NPAD, D), jnp.float32),
    compiler_params=pltpu.CompilerParams(use_tc_tiling_on_sc=False),
    scratch_types=[
        pltpu.VMEM((ECK,), jnp.int32),
        pltpu.VMEM((ECK, D), jnp.float32),
        pltpu.SemaphoreType.DMA,
    ],
)
def _emb_gather(table_h, gid_h, out_h, idx_v, rows_v, sem):
    c = lax.axis_index("c")
    s = lax.axis_index("s")
    wid = s * 2 + c
    base = wid * (NPAD // 32)
    for k in range(NPAD // 32 // ECK):
        pltpu.sync_copy(gid_h.at[pl.ds(base + k * ECK, ECK)], idx_v)
        pltpu.async_copy(table_h.at[idx_v], rows_v, sem).wait()
        pltpu.sync_copy(rows_v, out_h.at[pl.ds(base + k * ECK, ECK)])


@functools.partial(
    pl.kernel,
    mesh=_mesh(),
    out_type=jax.ShapeDtypeStruct((2, 16, HALFP, 16), jnp.float32),
    compiler_params=pltpu.CompilerParams(use_tc_tiling_on_sc=False),
    scratch_types=[
        pltpu.VMEM((2, CK), jnp.int32),
        pltpu.VMEM((2, CK), jnp.int32),
        pltpu.VMEM((2, CK), jnp.int32),
        pltpu.VMEM((2, CK), jnp.float32),
        pltpu.VMEM((2, CK), jnp.float32),
        pltpu.VMEM((2, CK), jnp.int32),
        pltpu.VMEM((2, CK, 16), jnp.float32),
        pltpu.VMEM((HALFP, 16), jnp.float32),
        pltpu.SemaphoreType.DMA,
        pltpu.SemaphoreType.DMA,
    ],
)
def _edge_pass(table_h, gidx_h, dst_h, co_h, out_h,
               idx_v, idx2_v, dst_v, co_v, w_v, sidx_v, rows_v, acc_v,
               sem0, sem1):
    c = lax.axis_index("c")
    s = lax.axis_index("s")
    base = c * HALF
    sems = (sem0, sem1)
    zero16 = jnp.zeros((16,), jnp.float32)

    def _zrow(r, carry):
        acc_v[r, :] = zero16
        return carry

    lax.fori_loop(0, HALFP, _zrow, 0)

    def _stage(ci, b):
        eb = ci * CK
        pltpu.sync_copy(gidx_h.at[pl.ds(eb, CK)], idx_v.at[b])
        pltpu.sync_copy(dst_h.at[pl.ds(eb, CK)], dst_v.at[b])
        pltpu.sync_copy(co_h.at[pl.ds(eb, CK)], co_v.at[b])
        for j in range(CK // 16):
            sl = pl.ds(j * 16, 16)
            idx2_v[b, sl] = idx_v[b, sl] * 16 + s
            d = dst_v[b, sl]
            inr = (d >= base) & (d < base + HALF)
            sidx_v[b, sl] = jnp.where(inr, d - base, HALF)
            w_v[b, sl] = jnp.where(inr, co_v[b, sl], 0.0)
        pltpu.async_copy(table_h.at[idx2_v.at[b]], rows_v.at[b], sems[b])

    def _drain_and_acc(b):
        pltpu.make_async_copy(
            table_h.at[idx2_v.at[b]], rows_v.at[b], sems[b]).wait()

        def _acc(g, cc):
            gsl = pl.ds(g * 16, 16)
            wv = w_v[b, gsl]
            sv = sidx_v[b, gsl]
            for rr in range(16):
                wsc = wv[rr]
                lr = sv[rr]
                r = g * 16 + rr
                plsc.addupdate(acc_v.at[lr], rows_v[b, r, :] * wsc)
            return cc

        lax.fori_loop(0, CK // 16, _acc, 0)

    _stage(0, 0)

    def _pair(cp, carry):
        for b in range(2):
            ci = cp * 2 + b

            @pl.when(ci + 1 < NCH)
            def _():
                _stage(ci + 1, 1 - b)

            _drain_and_acc(b)
        return carry

    lax.fori_loop(0, NCH // 2, _pair, 0)
    pltpu.sync_copy(acc_v, out_h.at[c, s])


def _input_body(agg_ref, emb_ref, spo_ref, wemb_ref, wspo_ref, b_ref, out_ref):
    mask = (spo_ref[:, 0:1] + spo_ref[:, 2:3]) > 0.0
    emb2 = jnp.where(mask, agg_ref[0], emb_ref[...])
    h = (jnp.dot(emb2, wemb_ref[...], preferred_element_type=jnp.float32)
         + jnp.dot(spo_ref[...], wspo_ref[...], preferred_element_type=jnp.float32)
         + b_ref[...])
    out_ref[...] = h


def _k_input(agg0, emb, spo_f, w_emb, w_spo, b_row):
    return pl.pallas_call(
        _input_body,
        grid=(B,),
        in_specs=[
            pl.BlockSpec((1, G, D), lambda i: (i // 5, i % 5, 0)),
            pl.BlockSpec((G, D), lambda i: (i, 0)),
            pl.BlockSpec((G, 3), lambda i: (i, 0)),
            pl.BlockSpec((D, D), lambda i: (0, 0)),
            pl.BlockSpec((3, D), lambda i: (0, 0)),
            pl.BlockSpec((1, D), lambda i: (0, 0)),
        ],
        out_specs=pl.BlockSpec((G, D), lambda i: (i, 0)),
        out_shape=jax.ShapeDtypeStruct((N, D), jnp.float32),
    )(agg0, emb, spo_f, w_emb, w_spo, b_row)


def _transform_body(h_ref, wrel_ref, wg_ref, bg_ref, out_ref):
    mm = jnp.dot(h_ref[...], wrel_ref[0], preferred_element_type=jnp.float32)
    g = jnp.dot(h_ref[...], wg_ref[0], preferred_element_type=jnp.float32)
    sgate = jax.nn.sigmoid(g[:, 0:1] + bg_ref[0, 0:1, 0:1])
    out_ref[0] = mm * sgate


def _k_transform(h, wrel, wg_pad, bg_pad):
    return pl.pallas_call(
        _transform_body,
        grid=(B, T),
        in_specs=[
            pl.BlockSpec((G, D), lambda i, t: (i, 0)),
            pl.BlockSpec((1, D, D), lambda i, t: (t, 0, 0)),
            pl.BlockSpec((1, D, 128), lambda i, t: (t, 0, 0)),
            pl.BlockSpec((1, 8, 128), lambda i, t: (t, 0, 0)),
        ],
        out_specs=pl.BlockSpec((1, G, D), lambda i, t: (t, i, 0)),
        out_shape=jax.ShapeDtypeStruct((T, N, D), jnp.float32),
    )(h, wrel, wg_pad, bg_pad)


def _relu_add_body(agg_ref, h0_ref, out_ref):
    out_ref[...] = jnp.maximum(agg_ref[0], 0.0) + h0_ref[...]


def _k_relu_add(parts, h0):
    return pl.pallas_call(
        _relu_add_body,
        grid=(B,),
        in_specs=[
            pl.BlockSpec((1, G, D), lambda i: (i // 5, i % 5, 0)),
            pl.BlockSpec((G, D), lambda i: (i, 0)),
        ],
        out_specs=pl.BlockSpec((G, D), lambda i: (i, 0)),
        out_shape=jax.ShapeDtypeStruct((N, D), jnp.float32),
    )(parts, h0)


def _pool_body(agg_ref, mb_ref, enc_ref):
    hf = jnp.maximum(agg_ref[0], 0.0)
    mb_ref[0] = hf
    enc_ref[0, 0] = jnp.sum(hf, axis=0) * (1.0 / G)


def _k_pool(parts):
    return pl.pallas_call(
        _pool_body,
        grid=(B,),
        in_specs=[pl.BlockSpec((1, G, D), lambda g: (g // 5, g % 5, 0))],
        out_specs=[
            pl.BlockSpec((1, G, D), lambda g: (g, 0, 0)),
            pl.BlockSpec((1, 1, D), lambda g: (g, 0, 0)),
        ],
        out_shape=[
            jax.ShapeDtypeStruct((B, G, D), jnp.float32),
            jax.ShapeDtypeStruct((B, 1, D), jnp.float32),
        ],
    )(parts)


def _heads_body(enc_ref, h_ref, o1_ref, o2_ref):
    e = enc_ref[...]
    o1_ref[0] = jnp.dot(e, h_ref[0], preferred_element_type=jnp.float32)
    o1_ref[1] = jnp.dot(e, h_ref[1], preferred_element_type=jnp.float32)
    o2_ref[0] = jnp.dot(e, h_ref[2], preferred_element_type=jnp.float32)
    o2_ref[1] = jnp.dot(e, h_ref[3], preferred_element_type=jnp.float32)


def _k_heads(enc, h4):
    return pl.pallas_call(
        _heads_body,
        out_shape=[
            jax.ShapeDtypeStruct((2, B, D), jnp.float32),
            jax.ShapeDtypeStruct((2, B, D), jnp.float32),
        ],
    )(enc, h4)


def kernel(global_id, spo, edge_src, edge_dst, edge_type, edge_norm,
           emb_table, W_et, b_et, H, Wrel, Wg, bg):
    gid = jnp.concatenate([global_id.astype(jnp.int32),
                           jnp.zeros((NPAD - N,), jnp.int32)])
    emb_full = _emb_gather(emb_table, gid)
    emb = emb_full[:N]

    src = edge_src.astype(jnp.int32)
    dst = edge_dst.astype(jnp.int32)
    norm = edge_norm[:, 0]
    coeff0 = jnp.where(edge_type == NE_T, norm, 0.0)
    parts0 = _edge_pass(emb.reshape(N * 16, 16), src, dst, coeff0)
    agg0 = parts0.transpose(0, 2, 1, 3).reshape(2, HALFP, D)

    spo_f = spo.astype(jnp.float32)
    w_emb = W_et[:D]
    w_spo = W_et[D:]
    b_row = b_et.reshape(1, D)
    h0 = _k_input(agg0, emb, spo_f, w_emb, w_spo, b_row)

    gidx = edge_type.astype(jnp.int32) * N + src
    wg_pad = jnp.pad(Wg, ((0, 0), (0, 0), (0, 0), (0, 127)))
    bg_pad = jnp.broadcast_to(bg.reshape(2, T, 1, 1), (2, T, 8, 128))
    htg0 = _k_transform(h0, Wrel[0], wg_pad[0], bg_pad[0])
    parts_l0 = _edge_pass(htg0.reshape(T * N * 16, 16), gidx, dst, norm)
    agg_l0 = parts_l0.transpose(0, 2, 1, 3).reshape(2, HALFP, D)
    h1 = _k_relu_add(agg_l0, h0)

    htg1 = _k_transform(h1, Wrel[1], wg_pad[1], bg_pad[1])
    parts_l1 = _edge_pass(htg1.reshape(T * N * 16, 16), gidx, dst, norm)
    agg_l1 = parts_l1.transpose(0, 2, 1, 3).reshape(2, HALFP, D)

    mb_raw, enc_raw = _k_pool(agg_l1)
    enc = enc_raw.reshape(B, D)
    h1o, h2o = _k_heads(enc, H)
    memory_bank = mb_raw.transpose(1, 0, 2)
    return (h1o, h2o, memory_bank)
